# baseline (speedup 1.0000x reference)
import functools

import jax
import jax.numpy as jnp
from jax import lax
from jax.experimental import pallas as pl
from jax.experimental.pallas import tpu as pltpu
from jax.experimental.pallas import tpu_sc as plsc


@functools.lru_cache(maxsize=None)
def _build_sc_kernel(n: int, m: int):
    info = plsc.get_sparse_core_info()
    num_cores, num_subcores, lanes = (
        info.num_cores,
        info.num_subcores,
        info.num_lanes,
    )
    num_workers = num_cores * num_subcores
    n_per_w = n // num_workers
    m_per_t = m // num_subcores
    idx_rows = m_per_t // 128
    half = n_per_w // 2
    assert n % num_workers == 0 and n_per_w % (2 * lanes) == 0
    assert m % num_subcores == 0 and m_per_t % 128 == 0

    mesh = plsc.VectorSubcoreMesh(core_axis_name="c", subcore_axis_name="s")

    @functools.partial(
        pl.kernel,
        mesh=mesh,
        out_type=jax.ShapeDtypeStruct((n,), jnp.int32),
        compiler_params=pltpu.CompilerParams(needs_layout_passes=False),
        scratch_types=[
            pltpu.VMEM((idx_rows, 128), jnp.int32),
            pltpu.VMEM((m_per_t,), jnp.int32),
            pltpu.VMEM_SHARED((m,), jnp.int32),
            pltpu.VMEM((n_per_w,), jnp.int32),
            pltpu.VMEM((n_per_w,), jnp.int32),
            pltpu.SemaphoreType.DMA,
            pltpu.SemaphoreType.DMA,
        ],
    )
    def sc_kernel(
        tensor_hbm,
        mapping_hbm,
        out_hbm,
        mapc_v,
        vals_v,
        inv_sh,
        t_v,
        o_v,
        sem0,
        sem1,
    ):
        sid = lax.axis_index("s")
        wid = sid * num_cores + lax.axis_index("c")
        base = wid * n_per_w

        t_cp = pltpu.async_copy(tensor_hbm.at[pl.ds(base, n_per_w)], t_v, sem1)
        map_cp = pltpu.async_copy(
            mapping_hbm.at[pl.ds(sid * idx_rows, idx_rows)], mapc_v, sem0
        )

        val_base = sid * m_per_t

        @plsc.parallel_loop(0, m_per_t // lanes, unroll=16)
        def mkvals(q):
            vals_v[pl.ds(q * lanes, lanes)] = (
                lax.iota(jnp.int32, lanes) + (val_base + q * lanes)
            )

        map_cp.wait()
        scatter_cps = [
            pltpu.async_copy(
                vals_v.at[pl.ds(j * 128, 128)],
                inv_sh.at[mapc_v.at[j]],
                sem0,
            )
            for j in range(idx_rows)
        ]
        for cp in scatter_cps:
            cp.wait()
        plsc.subcore_barrier()

        t_cp.wait()
        gather_cps = [
            pltpu.async_copy(
                inv_sh.at[t_v.at[pl.ds(j * 128, 128)]],
                o_v.at[pl.ds(j * 128, 128)],
                sem0,
            )
            for j in range(n_per_w // 128)
        ]
        for cp in gather_cps:
            cp.wait()
        pltpu.sync_copy(o_v, out_hbm.at[pl.ds(base, n_per_w)])

    return sc_kernel


def kernel(tensor, mapping):
    n = tensor.shape[0]
    m = mapping.shape[0]
    out = _build_sc_kernel(n, m)(
        tensor.astype(jnp.int32),
        mapping.astype(jnp.int32).reshape(m // 128, 128),
    )
    return out.astype(tensor.dtype)


# speedup vs baseline: 1.0241x; 1.0241x over previous
import functools

import jax
import jax.numpy as jnp
from jax import lax
from jax.experimental import pallas as pl
from jax.experimental.pallas import tpu as pltpu
from jax.experimental.pallas import tpu_sc as plsc


@functools.lru_cache(maxsize=None)
def _build_sc_kernel(n: int, m: int):
    info = plsc.get_sparse_core_info()
    num_cores, num_subcores, lanes = (
        info.num_cores,
        info.num_subcores,
        info.num_lanes,
    )
    num_workers = num_cores * num_subcores
    n_per_w = n // num_workers
    m_per_t = m // num_subcores
    idx_rows = m_per_t // 128
    half = n_per_w // 2
    assert n % num_workers == 0 and n_per_w % (2 * lanes) == 0
    assert m % num_subcores == 0 and m_per_t % 128 == 0

    mesh = plsc.VectorSubcoreMesh(core_axis_name="c", subcore_axis_name="s")

    @functools.partial(
        pl.kernel,
        mesh=mesh,
        out_type=jax.ShapeDtypeStruct((n,), jnp.int32),
        compiler_params=pltpu.CompilerParams(needs_layout_passes=False),
        scratch_types=[
            pltpu.VMEM((idx_rows, 128), jnp.int32),
            pltpu.VMEM((m_per_t,), jnp.int32),
            pltpu.VMEM_SHARED((m,), jnp.int32),
            pltpu.VMEM((m,), jnp.int32),
            pltpu.VMEM((n_per_w,), jnp.int32),
            pltpu.VMEM((n_per_w,), jnp.int32),
            pltpu.SemaphoreType.DMA,
            pltpu.SemaphoreType.DMA,
        ],
    )
    def sc_kernel(
        tensor_hbm,
        mapping_hbm,
        out_hbm,
        mapc_v,
        vals_v,
        inv_sh,
        inv_v,
        t_v,
        o_v,
        sem0,
        sem1,
    ):
        sid = lax.axis_index("s")
        wid = sid * num_cores + lax.axis_index("c")
        base = wid * n_per_w

        t_cp = pltpu.async_copy(tensor_hbm.at[pl.ds(base, n_per_w)], t_v, sem1)
        map_cp = pltpu.async_copy(
            mapping_hbm.at[pl.ds(sid * idx_rows, idx_rows)], mapc_v, sem0
        )

        val_base = sid * m_per_t

        @plsc.parallel_loop(0, m_per_t // lanes, unroll=16)
        def mkvals(q):
            vals_v[pl.ds(q * lanes, lanes)] = (
                lax.iota(jnp.int32, lanes) + (val_base + q * lanes)
            )

        map_cp.wait()
        scatter_cps = [
            pltpu.async_copy(
                vals_v.at[pl.ds(j * 128, 128)],
                inv_sh.at[mapc_v.at[j]],
                sem0,
            )
            for j in range(idx_rows)
        ]
        for cp in scatter_cps:
            cp.wait()
        plsc.subcore_barrier()

        inv_cp = pltpu.async_copy(inv_sh, inv_v, sem0)
        t_cp.wait()
        inv_cp.wait()

        @plsc.parallel_loop(0, half // lanes, unroll=8)
        def resolve_lo(i):
            idx = t_v[pl.ds(i * lanes, lanes)]
            o_v[pl.ds(i * lanes, lanes)] = plsc.load_gather(inv_v, [idx])

        out_lo_cp = pltpu.async_copy(
            o_v.at[pl.ds(0, half)], out_hbm.at[pl.ds(base, half)], sem1
        )

        @plsc.parallel_loop(half // lanes, n_per_w // lanes, unroll=8)
        def resolve_hi(i):
            idx = t_v[pl.ds(i * lanes, lanes)]
            o_v[pl.ds(i * lanes, lanes)] = plsc.load_gather(inv_v, [idx])

        out_lo_cp.wait()
        pltpu.sync_copy(
            o_v.at[pl.ds(half, half)], out_hbm.at[pl.ds(base + half, half)]
        )

    return sc_kernel


def kernel(tensor, mapping):
    n = tensor.shape[0]
    m = mapping.shape[0]
    out = _build_sc_kernel(n, m)(
        tensor.astype(jnp.int32),
        mapping.astype(jnp.int32).reshape(m // 128, 128),
    )
    return out.astype(tensor.dtype)
